# baseline (speedup 1.0000x reference)
import functools

import jax
import jax.numpy as jnp
from jax import lax
from jax.experimental import pallas as pl
from jax.experimental.pallas import tpu as pltpu
from jax.experimental.pallas import tpu_sc as plsc

B = 16384
F = 26
D = 13
VOCAB = 1000000

NC = 2
NS = 16
NW = NC * NS
BPW = B // NW
CPW = BPW // 128
SUB = 128 // 16

_mesh = plsc.VectorSubcoreMesh(core_axis_name="c", subcore_axis_name="s")


@functools.partial(
    pl.kernel,
    mesh=_mesh,
    out_type=jax.ShapeDtypeStruct((B,), jnp.float32),
    scratch_types=[
        pltpu.VMEM((F, CPW, 128), jnp.int32),
        pltpu.VMEM((F, CPW, 128), jnp.float32),
        pltpu.VMEM((D, CPW, 128), jnp.float32),
        pltpu.VMEM((D, 16), jnp.float32),
        pltpu.VMEM((BPW,), jnp.float32),
        pltpu.SemaphoreType.DMA,
    ],
)
def _fm_first_order(table_hbm, idx_hbm, dense_hbm, w_hbm, out_hbm,
                    idxv, vals, dv, wv, outv, gsem):
    w = lax.axis_index("s") * NC + lax.axis_index("c")

    pltpu.sync_copy(idx_hbm.at[w], idxv)
    pltpu.sync_copy(dense_hbm.at[w], dv)
    pltpu.sync_copy(w_hbm, wv)

    wregs = [wv[j, :] for j in range(D)]
    for c in range(CPW):
        for s in range(SUB):
            acc = dv[0, c, pl.ds(s * 16, 16)] * wregs[0]
            for j in range(1, D):
                acc = acc + dv[j, c, pl.ds(s * 16, 16)] * wregs[j]
            outv[pl.ds(c * 128 + s * 16, 16)] = acc

    def field_body(f, carry):
        off = f * VOCAB
        for c in range(CPW):
            for s in range(SUB):
                sl = pl.ds(s * 16, 16)
                idxv[f, c, sl] = idxv[f, c, sl] + off
        copies = [
            pltpu.async_copy(table_hbm.at[idxv.at[f, c]], vals.at[f, c], gsem)
            for c in range(CPW)
        ]
        for cp in copies:
            cp.wait()
        for c in range(CPW):
            for s in range(SUB):
                o = pl.ds(c * 128 + s * 16, 16)
                outv[o] = outv[o] + vals[f, c, pl.ds(s * 16, 16)]
        return carry

    lax.fori_loop(0, F, field_body, 0)

    pltpu.sync_copy(outv, out_hbm.at[pl.ds(w * BPW, BPW)])


def kernel(sparse_idx, dense, tables, dense_weights):
    idx4 = (sparse_idx.astype(jnp.int32)
            .reshape(NW, CPW, 128, F).transpose(0, 3, 1, 2))
    dense4 = dense.reshape(NW, CPW, 128, D).transpose(0, 3, 1, 2)
    w16 = jnp.tile(dense_weights, (1, 16))
    out = _fm_first_order(tables.reshape(-1), idx4, dense4, w16)
    return out.reshape(B, 1)


# speedup vs baseline: 1.0022x; 1.0022x over previous
import functools

import jax
import jax.numpy as jnp
from jax import lax
from jax.experimental import pallas as pl
from jax.experimental.pallas import tpu as pltpu
from jax.experimental.pallas import tpu_sc as plsc

B = 16384
F = 26
D = 13
VOCAB = 1000000

NC = 2
NS = 16
NW = NC * NS
BPW = B // NW
CPW = BPW // 128
SUB = 128 // 16

_mesh = plsc.VectorSubcoreMesh(core_axis_name="c", subcore_axis_name="s")


@functools.partial(
    pl.kernel,
    mesh=_mesh,
    out_type=jax.ShapeDtypeStruct((B,), jnp.float32),
    scratch_types=[
        pltpu.VMEM((F, CPW, 128), jnp.int32),
        pltpu.VMEM((CPW, 128), jnp.float32),
        pltpu.VMEM((D, CPW, 128), jnp.float32),
        pltpu.VMEM((D, 16), jnp.float32),
        pltpu.VMEM((BPW,), jnp.float32),
        pltpu.SemaphoreType.DMA,
    ],
)
def _fm_first_order(table_hbm, idx_hbm, dense_hbm, w_hbm, out_hbm,
                    idxv, vals, dv, wv, outv, gsem):
    w = lax.axis_index("s") * NC + lax.axis_index("c")

    pltpu.sync_copy(idx_hbm.at[w], idxv)
    pltpu.sync_copy(dense_hbm.at[w], dv)
    pltpu.sync_copy(w_hbm, wv)

    wregs = [wv[j, :] for j in range(D)]
    for c in range(CPW):
        for s in range(SUB):
            acc = dv[0, c, pl.ds(s * 16, 16)] * wregs[0]
            for j in range(1, D):
                acc = acc + dv[j, c, pl.ds(s * 16, 16)] * wregs[j]
            outv[pl.ds(c * 128 + s * 16, 16)] = acc

    def field_body(f, carry):
        copies = [
            pltpu.async_copy(table_hbm.at[idxv.at[f, c]], vals.at[c], gsem)
            for c in range(CPW)
        ]
        for cp in copies:
            cp.wait()
        for c in range(CPW):
            for s in range(SUB):
                o = pl.ds(c * 128 + s * 16, 16)
                outv[o] = outv[o] + vals[c, pl.ds(s * 16, 16)]
        return carry

    lax.fori_loop(0, F, field_body, 0)

    pltpu.sync_copy(outv, out_hbm.at[pl.ds(w * BPW, BPW)])


def kernel(sparse_idx, dense, tables, dense_weights):
    idx = sparse_idx.astype(jnp.int32) + (
        jnp.arange(F, dtype=jnp.int32)[None, :] * VOCAB)
    idx4 = idx.reshape(NW, BPW, F).transpose(0, 2, 1).reshape(NW, F, CPW, 128)
    dense4 = dense.reshape(NW, BPW, D).transpose(0, 2, 1).reshape(NW, D, CPW, 128)
    w16 = jnp.tile(dense_weights, (1, 16))
    out = _fm_first_order(tables.reshape(-1), idx4, dense4, w16)
    return out.reshape(B, 1)


# speedup vs baseline: 1.0088x; 1.0066x over previous
import functools

import jax
import jax.numpy as jnp
from jax import lax
from jax.experimental import pallas as pl
from jax.experimental.pallas import tpu as pltpu
from jax.experimental.pallas import tpu_sc as plsc

B = 16384
F = 26
D = 13
VOCAB = 1000000

NC = 2
NS = 16
NW = NC * NS
BPW = B // NW
CPW = BPW // 128
SUB = 128 // 16

_mesh = plsc.VectorSubcoreMesh(core_axis_name="c", subcore_axis_name="s")


@functools.partial(
    pl.kernel,
    mesh=_mesh,
    out_type=jax.ShapeDtypeStruct((B,), jnp.float32),
    compiler_params=pltpu.CompilerParams(use_tc_tiling_on_sc=False),
    scratch_types=[
        pltpu.VMEM((F, CPW, 128), jnp.int32),
        pltpu.VMEM((F, CPW, 128), jnp.float32),
        pltpu.VMEM((D, CPW, 128), jnp.float32),
        pltpu.VMEM((D, 16), jnp.float32),
        pltpu.VMEM((BPW,), jnp.float32),
        pltpu.SemaphoreType.DMA,
    ],
)
def _fm_first_order(table_hbm, idx_hbm, dense_hbm, w_hbm, out_hbm,
                    idxv, vals, dv, wv, outv, gsem):
    w = lax.axis_index("s") * NC + lax.axis_index("c")

    pltpu.sync_copy(idx_hbm.at[w], idxv)
    pltpu.sync_copy(dense_hbm.at[w], dv)
    pltpu.sync_copy(w_hbm, wv)

    copies = [
        pltpu.async_copy(table_hbm.at[f].at[idxv.at[f, c]], vals.at[f, c], gsem)
        for f in range(F)
        for c in range(CPW)
    ]

    wregs = [wv[j, :] for j in range(D)]
    for c in range(CPW):
        for s in range(SUB):
            acc = dv[0, c, pl.ds(s * 16, 16)] * wregs[0]
            for j in range(1, D):
                acc = acc + dv[j, c, pl.ds(s * 16, 16)] * wregs[j]
            outv[pl.ds(c * 128 + s * 16, 16)] = acc

    for cp in copies:
        cp.wait()

    for c in range(CPW):
        for s in range(SUB):
            o = pl.ds(c * 128 + s * 16, 16)
            acc = outv[o]
            for f in range(F):
                acc = acc + vals[f, c, pl.ds(s * 16, 16)]
            outv[o] = acc

    pltpu.sync_copy(outv, out_hbm.at[pl.ds(w * BPW, BPW)])


def kernel(sparse_idx, dense, tables, dense_weights):
    idx = sparse_idx.astype(jnp.int32)
    idx4 = idx.reshape(NW, BPW, F).transpose(0, 2, 1).reshape(NW, F, CPW, 128)
    dense4 = dense.reshape(NW, BPW, D).transpose(0, 2, 1).reshape(NW, D, CPW, 128)
    w16 = jnp.tile(dense_weights, (1, 16))
    out = _fm_first_order(tables, idx4, dense4, w16)
    return out.reshape(B, 1)


# speedup vs baseline: 1.0088x; 1.0000x over previous
import functools

import jax
import jax.numpy as jnp
from jax import lax
from jax.experimental import pallas as pl
from jax.experimental.pallas import tpu as pltpu
from jax.experimental.pallas import tpu_sc as plsc

B = 16384
F = 26
D = 13
VOCAB = 1000000

NC = 2
NS = 16
NW = NC * NS
BPW = B // NW
CPW = BPW // 128
SUB = 128 // 16

_mesh = plsc.VectorSubcoreMesh(core_axis_name="c", subcore_axis_name="s")


@functools.partial(
    pl.kernel,
    mesh=_mesh,
    out_type=jax.ShapeDtypeStruct((B,), jnp.float32),
    compiler_params=pltpu.CompilerParams(use_tc_tiling_on_sc=False),
    scratch_types=[
        pltpu.VMEM((F * CPW * 128,), jnp.int32),
        pltpu.VMEM((F * CPW * 128,), jnp.float32),
        pltpu.VMEM((D, CPW, 128), jnp.float32),
        pltpu.VMEM((D, 16), jnp.float32),
        pltpu.VMEM((BPW,), jnp.float32),
        pltpu.SemaphoreType.DMA,
    ],
)
def _fm_first_order(table_hbm, idx_hbm, dense_hbm, w_hbm, out_hbm,
                    idxv, vals, dv, wv, outv, gsem):
    w = lax.axis_index("s") * NC + lax.axis_index("c")

    pltpu.sync_copy(idx_hbm.at[w], idxv)
    pltpu.sync_copy(dense_hbm.at[w], dv)
    pltpu.sync_copy(w_hbm, wv)

    gather = pltpu.async_copy(table_hbm.at[idxv], vals, gsem)

    wregs = [wv[j, :] for j in range(D)]
    for c in range(CPW):
        for s in range(SUB):
            acc = dv[0, c, pl.ds(s * 16, 16)] * wregs[0]
            for j in range(1, D):
                acc = acc + dv[j, c, pl.ds(s * 16, 16)] * wregs[j]
            outv[pl.ds(c * 128 + s * 16, 16)] = acc

    gather.wait()

    for c in range(CPW):
        for s in range(SUB):
            o = pl.ds(c * 128 + s * 16, 16)
            acc = outv[o]
            for f in range(F):
                acc = acc + vals[pl.ds((f * CPW + c) * 128 + s * 16, 16)]
            outv[o] = acc

    pltpu.sync_copy(outv, out_hbm.at[pl.ds(w * BPW, BPW)])


def kernel(sparse_idx, dense, tables, dense_weights):
    flat_idx = sparse_idx.astype(jnp.int32) + jnp.arange(
        F, dtype=jnp.int32) * VOCAB
    idx4 = flat_idx.reshape(NW, BPW, F).transpose(0, 2, 1).reshape(
        NW, F * CPW * 128)
    dense4 = dense.reshape(NW, BPW, D).transpose(0, 2, 1).reshape(NW, D, CPW, 128)
    w16 = jnp.tile(dense_weights, (1, 16))
    out = _fm_first_order(tables.reshape(F * VOCAB), idx4, dense4, w16)
    return out.reshape(B, 1)
